# Initial kernel scaffold; baseline (speedup 1.0000x reference)
import jax
import jax.numpy as jnp
from jax.experimental import pallas as pl


def kernel(roi_feats, W_in, b_in, W_enc, b_enc, codebook, W_dec, b_dec, W_out, b_out):
    raise NotImplementedError("write your pallas kernel here")


# speedup vs baseline: 1.0242x; 1.0242x over previous
import jax
import jax.numpy as jnp
from jax import lax
from jax.experimental import pallas as pl

_DN = (((1,), (0,)), ((), ()))


def _mm_bias_kernel(x_ref, w_ref, b_ref, o_ref):
    o_ref[...] = lax.dot_general(
        x_ref[...], w_ref[...], _DN,
        preferred_element_type=jnp.float32) + b_ref[...]


def _mm_bias(x, w, b, bn=512):
    n, k = x.shape
    m = w.shape[1]
    return pl.pallas_call(
        _mm_bias_kernel,
        grid=(n // bn,),
        in_specs=[pl.BlockSpec((bn, k), lambda i: (i, 0)),
                  pl.BlockSpec((k, m), lambda i: (0, 0)),
                  pl.BlockSpec((1, m), lambda i: (0, 0))],
        out_specs=pl.BlockSpec((bn, m), lambda i: (i, 0)),
        out_shape=jax.ShapeDtypeStruct((n, m), jnp.float32),
    )(x, w, b.reshape(1, m))


def kernel(roi_feats, W_in, b_in, W_enc, b_enc, codebook, W_dec, b_dec, W_out, b_out):
    commitment_cost = 0.25
    z_e = _mm_bias(_mm_bias(roi_feats, W_in, b_in), W_enc, b_enc)
    z_e_flat = z_e.reshape(-1, z_e.shape[-1])
    d = (jnp.sum(z_e_flat ** 2, axis=1, keepdims=True)
         - 2.0 * (z_e_flat @ codebook.T)
         + jnp.sum(codebook ** 2, axis=1))
    min_encoding_indices = jnp.argmin(d, axis=-1)
    z_q = jnp.take(codebook, min_encoding_indices, axis=0)
    z_q = z_e + jax.lax.stop_gradient(z_q - z_e)
    decoded = _mm_bias(z_q, W_dec, b_dec)
    recon = _mm_bias(decoded, W_out, b_out)
    recon_loss = jnp.mean((recon - roi_feats) ** 2)
    embedding_loss = jnp.mean((jax.lax.stop_gradient(z_q) - z_e) ** 2)
    commitment_loss = jnp.mean((z_q - jax.lax.stop_gradient(z_e)) ** 2)
    vq_loss = recon_loss + embedding_loss + commitment_cost * commitment_loss
    return (min_encoding_indices, z_q, recon, vq_loss, recon_loss,
            embedding_loss, commitment_loss)


# speedup vs baseline: 1.2335x; 1.2045x over previous
import functools

import jax
import jax.numpy as jnp
from jax import lax
from jax.experimental import pallas as pl
from jax.experimental.pallas import tpu as pltpu
from jax.experimental.pallas import tpu_sc as plsc

_DN = (((1,), (0,)), ((), ()))


def _enc_kernel(x_ref, wi_ref, bi_ref, we_ref, be_ref, o_ref):
    h = lax.dot_general(x_ref[...], wi_ref[...], _DN,
                        preferred_element_type=jnp.float32) + bi_ref[...]
    o_ref[...] = lax.dot_general(h, we_ref[...], _DN,
                                 preferred_element_type=jnp.float32) + be_ref[...]


def _encode(roi, W_in, b_in, W_enc, b_enc, bn=512):
    n, k = roi.shape
    m = W_in.shape[1]
    return pl.pallas_call(
        _enc_kernel,
        grid=(n // bn,),
        in_specs=[pl.BlockSpec((bn, k), lambda i: (i, 0)),
                  pl.BlockSpec((k, m), lambda i: (0, 0)),
                  pl.BlockSpec((1, m), lambda i: (0, 0)),
                  pl.BlockSpec((m, m), lambda i: (0, 0)),
                  pl.BlockSpec((1, m), lambda i: (0, 0))],
        out_specs=pl.BlockSpec((bn, m), lambda i: (i, 0)),
        out_shape=jax.ShapeDtypeStruct((n, m), jnp.float32),
    )(roi, W_in, b_in.reshape(1, m), W_enc, b_enc.reshape(1, m))


def _sc_gather(table, idx):
    info = plsc.get_sparse_core_info()
    nw = info.num_cores * info.num_subcores
    b = idx.shape[0]
    d = table.shape[1]
    b_per_w = b // nw
    chunk = 64
    mesh = plsc.VectorSubcoreMesh(core_axis_name="c", subcore_axis_name="s")

    @functools.partial(
        pl.kernel, mesh=mesh,
        out_type=jax.ShapeDtypeStruct((b, d), jnp.float32),
        scratch_types=[pltpu.VMEM((chunk,), jnp.int32),
                       pltpu.VMEM((chunk, d), jnp.float32),
                       pltpu.SemaphoreType.DMA],
    )
    def gather_k(table_hbm, idx_hbm, out_hbm, idx_v, rows_v, sem):
        wid = lax.axis_index("s") * info.num_cores + lax.axis_index("c")
        base = wid * b_per_w
        for c in range(b_per_w // chunk):
            off = base + c * chunk
            pltpu.sync_copy(idx_hbm.at[pl.ds(off, chunk)], idx_v)
            pltpu.async_copy(table_hbm.at[idx_v], rows_v, sem).wait()
            pltpu.sync_copy(rows_v, out_hbm.at[pl.ds(off, chunk)])

    return gather_k(table, idx)


def _dec_kernel(zq_ref, ze_ref, roi_ref, wd_ref, bd_ref, wo_ref, bo_ref,
                recon_ref, part_ref):
    zq = zq_ref[...]
    decoded = lax.dot_general(zq, wd_ref[...], _DN,
                              preferred_element_type=jnp.float32) + bd_ref[...]
    recon = lax.dot_general(decoded, wo_ref[...], _DN,
                            preferred_element_type=jnp.float32) + bo_ref[...]
    recon_ref[...] = recon
    sq = jnp.sum((recon - roi_ref[...]) ** 2)
    emb_d = zq - ze_ref[...]
    emb = jnp.sum(emb_d * emb_d)
    lanes = lax.broadcasted_iota(jnp.int32, (1, 1, 128), 2)
    part_ref[...] = (jnp.where(lanes == 0, sq, 0.0)
                     + jnp.where(lanes == 1, emb, 0.0))


def _decode(z_q, z_e, roi, W_dec, b_dec, W_out, b_out, bn=256):
    n, m = z_q.shape
    k = W_out.shape[1]
    nb = n // bn
    recon, parts = pl.pallas_call(
        _dec_kernel,
        grid=(nb,),
        in_specs=[pl.BlockSpec((bn, m), lambda i: (i, 0)),
                  pl.BlockSpec((bn, m), lambda i: (i, 0)),
                  pl.BlockSpec((bn, k), lambda i: (i, 0)),
                  pl.BlockSpec((m, m), lambda i: (0, 0)),
                  pl.BlockSpec((1, m), lambda i: (0, 0)),
                  pl.BlockSpec((m, k), lambda i: (0, 0)),
                  pl.BlockSpec((1, k), lambda i: (0, 0))],
        out_specs=[pl.BlockSpec((bn, k), lambda i: (i, 0)),
                   pl.BlockSpec((1, 1, 128), lambda i: (i, 0, 0))],
        out_shape=[jax.ShapeDtypeStruct((n, k), jnp.float32),
                   jax.ShapeDtypeStruct((nb, 1, 128), jnp.float32)],
    )(z_q, z_e, roi, W_dec, b_dec.reshape(1, m), W_out, b_out.reshape(1, k))
    return recon, parts


def kernel(roi_feats, W_in, b_in, W_enc, b_enc, codebook, W_dec, b_dec, W_out, b_out):
    commitment_cost = 0.25
    z_e = _encode(roi_feats, W_in, b_in, W_enc, b_enc)
    z_e_flat = z_e.reshape(-1, z_e.shape[-1])
    d = (jnp.sum(z_e_flat ** 2, axis=1, keepdims=True)
         - 2.0 * (z_e_flat @ codebook.T)
         + jnp.sum(codebook ** 2, axis=1))
    min_encoding_indices = jnp.argmin(d, axis=-1)
    z_q = _sc_gather(codebook, min_encoding_indices)
    recon, parts = _decode(z_q, z_e, roi_feats, W_dec, b_dec, W_out, b_out)
    n, k = roi_feats.shape
    m = z_e.shape[1]
    recon_loss = jnp.sum(parts[:, 0, 0]) / (n * k)
    embedding_loss = jnp.sum(parts[:, 0, 1]) / (n * m)
    commitment_loss = embedding_loss
    vq_loss = recon_loss + embedding_loss + commitment_cost * commitment_loss
    return (min_encoding_indices, z_q, recon, vq_loss, recon_loss,
            embedding_loss, commitment_loss)
